# baseline (speedup 1.0000x reference)
import functools

import jax
import jax.numpy as jnp
from jax import lax
from jax.experimental import pallas as pl
from jax.experimental.pallas import tpu as pltpu
from jax.experimental.pallas import tpu_sc as plsc

VOCAB = 1000000
DIM = 64
BATCH = 16384

_NC = 2
_NS = 16
_NW = _NC * _NS
_BPW = BATCH // _NW
_CHUNK = 128
_NCHUNK = _BPW // _CHUNK


def _sc_gather(table, hidx2d, tidx2d):
    mesh = plsc.VectorSubcoreMesh(core_axis_name="c", subcore_axis_name="s")

    @functools.partial(
        pl.kernel,
        mesh=mesh,
        out_type=[
            jax.ShapeDtypeStruct((BATCH, DIM), jnp.float32),
            jax.ShapeDtypeStruct((BATCH, DIM), jnp.float32),
        ],
        scratch_types=[
            pltpu.VMEM((_NCHUNK, _CHUNK), jnp.int32),
            pltpu.VMEM((_NCHUNK, _CHUNK), jnp.int32),
            pltpu.VMEM((_BPW, DIM), jnp.float32),
            pltpu.VMEM((_BPW, DIM), jnp.float32),
            pltpu.SemaphoreType.DMA,
            pltpu.SemaphoreType.DMA,
        ],
        compiler_params=pltpu.CompilerParams(use_tc_tiling_on_sc=False),
    )
    def body(table_hbm, hidx_hbm, tidx_hbm, hg_hbm, tg_hbm,
             hidx_v, tidx_v, hrows_v, trows_v, hsem, tsem):
        wid = lax.axis_index("s") * _NC + lax.axis_index("c")
        base = wid * _BPW
        row4 = wid * _NCHUNK
        pltpu.sync_copy(hidx_hbm.at[pl.ds(row4, _NCHUNK)], hidx_v)
        pltpu.sync_copy(tidx_hbm.at[pl.ds(row4, _NCHUNK)], tidx_v)
        hcopies = [
            pltpu.async_copy(
                table_hbm.at[hidx_v.at[c]],
                hrows_v.at[pl.ds(c * _CHUNK, _CHUNK)], hsem)
            for c in range(_NCHUNK)
        ]
        tcopies = [
            pltpu.async_copy(
                table_hbm.at[tidx_v.at[c]],
                trows_v.at[pl.ds(c * _CHUNK, _CHUNK)], tsem)
            for c in range(_NCHUNK)
        ]
        for cp in hcopies:
            cp.wait()
        for cp in tcopies:
            cp.wait()
        pltpu.sync_copy(hrows_v, hg_hbm.at[pl.ds(base, _BPW)])
        pltpu.sync_copy(trows_v, tg_hbm.at[pl.ds(base, _BPW)])

    return body(table, hidx2d, tidx2d)


def _mlp_body(h_ref, t_ref, a_ref, b_ref, b1_ref, w2t_ref, b2_ref, o_ref):
    x = (jnp.dot(h_ref[...], a_ref[...], preferred_element_type=jnp.float32)
         + jnp.dot(t_ref[...], b_ref[...], preferred_element_type=jnp.float32)
         + b1_ref[...])
    x = jnp.maximum(x, 0.0)
    o_ref[...] = (jnp.dot(x, w2t_ref[...], preferred_element_type=jnp.float32)
                  + b2_ref[...])


def _tc_mlp(hg, tg, w1h_t, w1t_t, b1, w2t, b2):
    blk = 2048
    grid = BATCH // blk
    return pl.pallas_call(
        _mlp_body,
        grid=(grid,),
        in_specs=[
            pl.BlockSpec((blk, DIM), lambda i: (i, 0)),
            pl.BlockSpec((blk, DIM), lambda i: (i, 0)),
            pl.BlockSpec((DIM, DIM), lambda i: (0, 0)),
            pl.BlockSpec((DIM, DIM), lambda i: (0, 0)),
            pl.BlockSpec((1, DIM), lambda i: (0, 0)),
            pl.BlockSpec((DIM, 2), lambda i: (0, 0)),
            pl.BlockSpec((1, 2), lambda i: (0, 0)),
        ],
        out_specs=pl.BlockSpec((blk, 2), lambda i: (i, 0)),
        out_shape=jax.ShapeDtypeStruct((BATCH, 2), jnp.float32),
    )(hg, tg, w1h_t, w1t_t, b1, w2t, b2)


def kernel(heads, tails, entity_embeddings, W1, b1, W2, b2):
    hidx2d = heads.astype(jnp.int32).reshape(BATCH // _CHUNK, _CHUNK)
    tidx2d = tails.astype(jnp.int32).reshape(BATCH // _CHUNK, _CHUNK)
    hg, tg = _sc_gather(entity_embeddings, hidx2d, tidx2d)
    w1t = W1.T
    w1h_t = w1t[:DIM]
    w1t_t = w1t[DIM:]
    return _tc_mlp(hg, tg, w1h_t, w1t_t, b1.reshape(1, DIM),
                   W2.T, b2.reshape(1, 2))
